# baseline (speedup 1.0000x reference)
import jax
import jax.numpy as jnp
from jax.experimental import pallas as pl
from jax.experimental.pallas import tpu as pltpu

_G = 128
_BN = 512
_CORES = 2


def _body(xb, bb, w0, b0, w1, b1, w2, b2, sums_ref, cnt_ref):
    j = pl.program_id(1)
    h = jnp.maximum(
        jnp.dot(xb[...], w0[...], preferred_element_type=jnp.float32) + b0[...], 0.0)
    h = jnp.maximum(
        jnp.dot(h.astype(jnp.bfloat16), w1[...],
                preferred_element_type=jnp.float32) + b1[...], 0.0)
    h = jnp.maximum(
        jnp.dot(h.astype(jnp.bfloat16), w2[...],
                preferred_element_type=jnp.float32) + b2[...], 0.0)
    ids = bb[0, 0, :]
    gids = jax.lax.broadcasted_iota(jnp.int32, (_G, _BN), 0)
    hits = gids == ids[None, :]
    contrib = jnp.dot(hits.astype(jnp.bfloat16), h.astype(jnp.bfloat16),
                      preferred_element_type=jnp.float32)
    cnt = jnp.sum(hits.astype(jnp.float32), axis=1, keepdims=True)
    cnt = jnp.broadcast_to(cnt, (_G, 128))

    @pl.when(j == 0)
    def _init():
        sums_ref[0] = contrib
        cnt_ref[0] = cnt

    @pl.when(j > 0)
    def _acc():
        sums_ref[0] += contrib
        cnt_ref[0] += cnt


def kernel(x, edge_index, batch, W0, b0, W1, b1, W2, b2):
    n, d_in = x.shape
    d_h = W0.shape[1]
    d_out = W2.shape[1]

    nb = pl.cdiv(n, _BN * _CORES) * _CORES
    npad = nb * _BN
    xp = jnp.pad(x.astype(jnp.bfloat16), ((0, npad - n), (0, 0)))
    ids = jnp.pad(batch.astype(jnp.int32), (0, npad - n), constant_values=_G)
    ids = ids.reshape(nb, 1, _BN)
    nbc = nb // _CORES

    sums, cnts = pl.pallas_call(
        _body,
        grid=(_CORES, nbc),
        in_specs=[
            pl.BlockSpec((_BN, d_in), lambda c, j: (c * nbc + j, 0)),
            pl.BlockSpec((1, 1, _BN), lambda c, j: (c * nbc + j, 0, 0)),
            pl.BlockSpec((d_in, d_h), lambda c, j: (0, 0)),
            pl.BlockSpec((1, d_h), lambda c, j: (0, 0)),
            pl.BlockSpec((d_h, d_h), lambda c, j: (0, 0)),
            pl.BlockSpec((1, d_h), lambda c, j: (0, 0)),
            pl.BlockSpec((d_h, d_out), lambda c, j: (0, 0)),
            pl.BlockSpec((1, d_out), lambda c, j: (0, 0)),
        ],
        out_specs=[
            pl.BlockSpec((1, _G, d_out), lambda c, j: (c, 0, 0)),
            pl.BlockSpec((1, _G, 128), lambda c, j: (c, 0, 0)),
        ],
        out_shape=[
            jax.ShapeDtypeStruct((_CORES, _G, d_out), jnp.float32),
            jax.ShapeDtypeStruct((_CORES, _G, 128), jnp.float32),
        ],
        compiler_params=pltpu.CompilerParams(
            dimension_semantics=("parallel", "arbitrary"),
            vmem_limit_bytes=56 * 1024 * 1024,
        ),
        name="gnnstack_fused",
    )(xp, ids, W0.astype(jnp.bfloat16), b0.reshape(1, d_h),
      W1.astype(jnp.bfloat16), b1.reshape(1, d_h),
      W2.astype(jnp.bfloat16), b2.reshape(1, d_out))

    total = sums.sum(axis=0)
    count = cnts[:, :, 0].sum(axis=0)
    return total / count[:, None]


# speedup vs baseline: 1.0579x; 1.0579x over previous
import jax
import jax.numpy as jnp
from jax.experimental import pallas as pl
from jax.experimental.pallas import tpu as pltpu

_G = 128
_BN = 512
_CORES = 2


def _body(xb, bb, w0, b0, w1, b1, w2, b2, sums_ref, cnt_ref):
    j = pl.program_id(1)
    h = jnp.maximum(
        jnp.dot(xb[...], w0[...], preferred_element_type=jnp.float32) + b0[...], 0.0)
    h = jnp.maximum(
        jnp.dot(h, w1[...], preferred_element_type=jnp.float32) + b1[...], 0.0)
    h = jnp.maximum(
        jnp.dot(h, w2[...], preferred_element_type=jnp.float32) + b2[...], 0.0)
    ids = bb[0, 0, :]
    gids = jax.lax.broadcasted_iota(jnp.int32, (_G, _BN), 0)
    hits = gids == ids[None, :]
    contrib = jnp.dot(hits.astype(jnp.float32), h,
                      preferred_element_type=jnp.float32)
    cnt = jnp.sum(hits.astype(jnp.float32), axis=1, keepdims=True)
    cnt = jnp.broadcast_to(cnt, (_G, 128))

    @pl.when(j == 0)
    def _init():
        sums_ref[0] = contrib
        cnt_ref[0] = cnt

    @pl.when(j > 0)
    def _acc():
        sums_ref[0] += contrib
        cnt_ref[0] += cnt


def kernel(x, edge_index, batch, W0, b0, W1, b1, W2, b2):
    n, d_in = x.shape
    d_h = W0.shape[1]
    d_out = W2.shape[1]

    nb = pl.cdiv(n, _BN * _CORES) * _CORES
    npad = nb * _BN
    xp = jnp.pad(x, ((0, npad - n), (0, 0)))
    ids = jnp.pad(batch.astype(jnp.int32), (0, npad - n), constant_values=_G)
    ids = ids.reshape(nb, 1, _BN)
    nbc = nb // _CORES

    sums, cnts = pl.pallas_call(
        _body,
        grid=(_CORES, nbc),
        in_specs=[
            pl.BlockSpec((_BN, d_in), lambda c, j: (c * nbc + j, 0)),
            pl.BlockSpec((1, 1, _BN), lambda c, j: (c * nbc + j, 0, 0)),
            pl.BlockSpec((d_in, d_h), lambda c, j: (0, 0)),
            pl.BlockSpec((1, d_h), lambda c, j: (0, 0)),
            pl.BlockSpec((d_h, d_h), lambda c, j: (0, 0)),
            pl.BlockSpec((1, d_h), lambda c, j: (0, 0)),
            pl.BlockSpec((d_h, d_out), lambda c, j: (0, 0)),
            pl.BlockSpec((1, d_out), lambda c, j: (0, 0)),
        ],
        out_specs=[
            pl.BlockSpec((1, _G, d_out), lambda c, j: (c, 0, 0)),
            pl.BlockSpec((1, _G, 128), lambda c, j: (c, 0, 0)),
        ],
        out_shape=[
            jax.ShapeDtypeStruct((_CORES, _G, d_out), jnp.float32),
            jax.ShapeDtypeStruct((_CORES, _G, 128), jnp.float32),
        ],
        compiler_params=pltpu.CompilerParams(
            dimension_semantics=("parallel", "arbitrary"),
            vmem_limit_bytes=56 * 1024 * 1024,
        ),
        name="gnnstack_fused",
    )(xp, ids, W0, b0.reshape(1, d_h), W1, b1.reshape(1, d_h),
      W2, b2.reshape(1, d_out))

    total = sums.sum(axis=0)
    count = cnts[:, :, 0].sum(axis=0)
    return total / count[:, None]


# speedup vs baseline: 1.0682x; 1.0097x over previous
import jax
import jax.numpy as jnp
from jax.experimental import pallas as pl
from jax.experimental.pallas import tpu as pltpu

_G = 128
_BN = 512


def _body(xb, bb, w0, b0, w1, b1, w2, b2, out_ref, sums_ref, cnt_ref):
    j = pl.program_id(0)
    nsteps = pl.num_programs(0)
    h = jnp.maximum(
        jnp.dot(xb[...], w0[...], preferred_element_type=jnp.float32) + b0[...], 0.0)
    h = jnp.maximum(
        jnp.dot(h, w1[...], preferred_element_type=jnp.float32) + b1[...], 0.0)
    h = jnp.maximum(
        jnp.dot(h, w2[...], preferred_element_type=jnp.float32) + b2[...], 0.0)
    ids = bb[0, 0, :]
    gids = jax.lax.broadcasted_iota(jnp.int32, (_G, _BN), 0)
    hits = (gids == ids[None, :]).astype(jnp.float32)
    contrib = jnp.dot(hits, h, preferred_element_type=jnp.float32)
    cnt = jnp.sum(hits, axis=1, keepdims=True)
    cnt = jnp.broadcast_to(cnt, (_G, 128))

    @pl.when(j == 0)
    def _init():
        sums_ref[...] = contrib
        cnt_ref[...] = cnt

    @pl.when(j > 0)
    def _acc():
        sums_ref[...] += contrib
        cnt_ref[...] += cnt

    @pl.when(j == nsteps - 1)
    def _finish():
        out_ref[...] = sums_ref[...] / cnt_ref[:, 0:1]


def kernel(x, edge_index, batch, W0, b0, W1, b1, W2, b2):
    n, d_in = x.shape
    d_h = W0.shape[1]
    d_out = W2.shape[1]

    nb = pl.cdiv(n, _BN)
    npad = nb * _BN
    xp = jnp.pad(x, ((0, npad - n), (0, 0)))
    ids = jnp.pad(batch.astype(jnp.int32), (0, npad - n), constant_values=_G)
    ids = ids.reshape(nb, 1, _BN)

    return pl.pallas_call(
        _body,
        grid=(nb,),
        in_specs=[
            pl.BlockSpec((_BN, d_in), lambda j: (j, 0)),
            pl.BlockSpec((1, 1, _BN), lambda j: (j, 0, 0)),
            pl.BlockSpec((d_in, d_h), lambda j: (0, 0)),
            pl.BlockSpec((1, d_h), lambda j: (0, 0)),
            pl.BlockSpec((d_h, d_h), lambda j: (0, 0)),
            pl.BlockSpec((1, d_h), lambda j: (0, 0)),
            pl.BlockSpec((d_h, d_out), lambda j: (0, 0)),
            pl.BlockSpec((1, d_out), lambda j: (0, 0)),
        ],
        out_specs=pl.BlockSpec((_G, d_out), lambda j: (0, 0)),
        out_shape=jax.ShapeDtypeStruct((_G, d_out), jnp.float32),
        scratch_shapes=[
            pltpu.VMEM((_G, d_out), jnp.float32),
            pltpu.VMEM((_G, 128), jnp.float32),
        ],
        compiler_params=pltpu.CompilerParams(
            dimension_semantics=("arbitrary",),
            vmem_limit_bytes=56 * 1024 * 1024,
        ),
        name="gnnstack_fused",
    )(xp, ids, W0, b0.reshape(1, d_h), W1, b1.reshape(1, d_h),
      W2, b2.reshape(1, d_out))


# speedup vs baseline: 1.1459x; 1.0727x over previous
import jax
import jax.numpy as jnp
from jax.experimental import pallas as pl
from jax.experimental.pallas import tpu as pltpu

_G = 128
_BN = 1024


def _body(xb, bb, w0, b0, w1, b1, w2, b2, out_ref, sums_ref, cnt_ref):
    j = pl.program_id(0)
    nsteps = pl.num_programs(0)
    h = jnp.maximum(
        jnp.dot(xb[...], w0[...], preferred_element_type=jnp.float32) + b0[...], 0.0)
    h = jnp.maximum(
        jnp.dot(h, w1[...], preferred_element_type=jnp.float32) + b1[...], 0.0)
    h = jnp.maximum(
        jnp.dot(h, w2[...], preferred_element_type=jnp.float32) + b2[...], 0.0)
    ids = bb[0, 0, :]
    gids = jax.lax.broadcasted_iota(jnp.int32, (_G, _BN), 0)
    hits = (gids == ids[None, :]).astype(jnp.float32)
    contrib = jnp.dot(hits, h, preferred_element_type=jnp.float32)
    cnt = jnp.sum(hits, axis=1, keepdims=True)
    cnt = jnp.broadcast_to(cnt, (_G, 128))

    @pl.when(j == 0)
    def _init():
        sums_ref[...] = contrib
        cnt_ref[...] = cnt

    @pl.when(j > 0)
    def _acc():
        sums_ref[...] += contrib
        cnt_ref[...] += cnt

    @pl.when(j == nsteps - 1)
    def _finish():
        out_ref[...] = sums_ref[...] / cnt_ref[:, 0:1]


def kernel(x, edge_index, batch, W0, b0, W1, b1, W2, b2):
    n, d_in = x.shape
    d_h = W0.shape[1]
    d_out = W2.shape[1]

    nb = pl.cdiv(n, _BN)
    npad = nb * _BN
    xp = jnp.pad(x, ((0, npad - n), (0, 0)))
    ids = jnp.pad(batch.astype(jnp.int32), (0, npad - n), constant_values=_G)
    ids = ids.reshape(nb, 1, _BN)

    return pl.pallas_call(
        _body,
        grid=(nb,),
        in_specs=[
            pl.BlockSpec((_BN, d_in), lambda j: (j, 0)),
            pl.BlockSpec((1, 1, _BN), lambda j: (j, 0, 0)),
            pl.BlockSpec((d_in, d_h), lambda j: (0, 0)),
            pl.BlockSpec((1, d_h), lambda j: (0, 0)),
            pl.BlockSpec((d_h, d_h), lambda j: (0, 0)),
            pl.BlockSpec((1, d_h), lambda j: (0, 0)),
            pl.BlockSpec((d_h, d_out), lambda j: (0, 0)),
            pl.BlockSpec((1, d_out), lambda j: (0, 0)),
        ],
        out_specs=pl.BlockSpec((_G, d_out), lambda j: (0, 0)),
        out_shape=jax.ShapeDtypeStruct((_G, d_out), jnp.float32),
        scratch_shapes=[
            pltpu.VMEM((_G, d_out), jnp.float32),
            pltpu.VMEM((_G, 128), jnp.float32),
        ],
        compiler_params=pltpu.CompilerParams(
            dimension_semantics=("arbitrary",),
            vmem_limit_bytes=56 * 1024 * 1024,
        ),
        name="gnnstack_fused",
    )(xp, ids, W0, b0.reshape(1, d_h), W1, b1.reshape(1, d_h),
      W2, b2.reshape(1, d_out))
